# baseline (speedup 1.0000x reference)
import jax
import jax.numpy as jnp
from jax import lax
from jax.experimental import pallas as pl
from jax.experimental.pallas import tpu as pltpu
from jax.experimental.pallas import tpu_sc as plsc

_BINS = (1, 2, 3, 4, 8, 16, 32, 64, 128, 256, 384)

_B = 16384
_D = 20
_NC, _NS, _L = 2, 16, 16
_NW = _NC * _NS
_BPW = _B // _NW
_RPI = 16
_CHR = 64


def _body(len_hbm, tab_hbm, out_hbm, len_v, gofs_v, tab_v, out_v, sem):
    wid = lax.axis_index("s") * _NC + lax.axis_index("c")
    base = wid * _BPW
    pltpu.sync_copy(len_hbm.at[pl.ds(base, _BPW)], len_v)
    pltpu.sync_copy(tab_hbm, tab_v)
    for j in range(_BPW // _L):
        v = len_v[pl.ds(j * _L, _L)]
        acc = lax.shift_right_logical(_BINS[0] - v, 31)
        for b in _BINS[1:]:
            acc = acc + lax.shift_right_logical(b - v, 31)
        gofs_v[pl.ds(j * _L, _L)] = acc * _D
    lane = lax.iota(jnp.int32, _L)

    def blk(i, carry):
        r0 = i * _RPI
        vrow = gofs_v[pl.ds(r0, _L)]
        for r in range(_RPI):
            g = vrow[r] + lane
            out_v[r0 + r, pl.ds(0, _L)] = plsc.load_gather(tab_v, [g])
            out_v[r0 + r, pl.ds(_D - _L, _L)] = plsc.load_gather(
                tab_v, [g + (_D - _L)]
            )
        return carry

    copies = []
    for c in range(_BPW // _CHR):
        lax.fori_loop(c * (_CHR // _RPI), (c + 1) * (_CHR // _RPI), blk, 0)
        copies.append(
            pltpu.async_copy(
                out_v.at[pl.ds(c * _CHR, _CHR)],
                out_hbm.at[pl.ds(base + c * _CHR, _CHR)],
                sem,
            )
        )
    for cp in copies:
        cp.wait()


def kernel(lengths, table):
    mesh = plsc.VectorSubcoreMesh(core_axis_name="c", subcore_axis_name="s")
    return pl.kernel(
        _body,
        out_type=jax.ShapeDtypeStruct((_B, _D), jnp.float32),
        mesh=mesh,
        scratch_types=[
            pltpu.VMEM((_BPW,), jnp.int32),
            pltpu.VMEM((_BPW,), jnp.int32),
            pltpu.VMEM((_D * 12,), jnp.float32),
            pltpu.VMEM((_BPW, _D), jnp.float32),
            pltpu.SemaphoreType.DMA,
        ],
        compiler_params=pltpu.CompilerParams(needs_layout_passes=False),
    )(lengths, table.reshape(-1))


# speedup vs baseline: 1.0407x; 1.0407x over previous
import jax
import jax.numpy as jnp
from jax import lax
from jax.experimental import pallas as pl
from jax.experimental.pallas import tpu as pltpu
from jax.experimental.pallas import tpu_sc as plsc

_BINS = (1, 2, 3, 4, 8, 16, 32, 64, 128, 256, 384)

_B = 16384
_D = 20
_NC, _NS, _L = 2, 16, 16
_NW = _NC * _NS
_BPW = _B // _NW
_RPI = 16


def _body(len_hbm, tab_hbm, out_hbm, len_v, idx_v, tab_v, out_v):
    wid = lax.axis_index("s") * _NC + lax.axis_index("c")
    base = wid * _BPW
    pltpu.sync_copy(len_hbm.at[pl.ds(base, _BPW)], len_v)
    pltpu.sync_copy(tab_hbm, tab_v)
    for j in range(_BPW // _L):
        v = len_v[pl.ds(j * _L, _L)]
        acc = lax.shift_right_logical(_BINS[0] - v, 31)
        for b in _BINS[1:]:
            acc = acc + lax.shift_right_logical(b - v, 31)
        idx_v[pl.ds(j * _L, _L)] = acc
    lane = lax.iota(jnp.int32, _L)
    lane4 = lane + (_D - _L)

    def blk(i, carry):
        r0 = i * _RPI
        vrow = idx_v[pl.ds(r0, _L)]
        for r in range(_RPI):
            rowv = jnp.full((_L,), vrow[r], jnp.int32)
            out_v[r0 + r, pl.ds(0, _L)] = plsc.load_gather(tab_v, [rowv, lane])
            out_v[r0 + r, pl.ds(_D - _L, _L)] = plsc.load_gather(
                tab_v, [rowv, lane4]
            )
        return carry

    lax.fori_loop(0, _BPW // _RPI, blk, 0)
    pltpu.sync_copy(out_v, out_hbm.at[pl.ds(base, _BPW)])


def kernel(lengths, table):
    mesh = plsc.VectorSubcoreMesh(core_axis_name="c", subcore_axis_name="s")
    return pl.kernel(
        _body,
        out_type=jax.ShapeDtypeStruct((_B, _D), jnp.float32),
        mesh=mesh,
        scratch_types=[
            pltpu.VMEM((_BPW,), jnp.int32),
            pltpu.VMEM((_BPW,), jnp.int32),
            pltpu.VMEM((12, _D), jnp.float32),
            pltpu.VMEM((_BPW, _D), jnp.float32),
        ],
        compiler_params=pltpu.CompilerParams(needs_layout_passes=False),
    )(lengths, table)


# speedup vs baseline: 1.2483x; 1.1995x over previous
import jax
import jax.numpy as jnp
from jax import lax
from jax.experimental import pallas as pl
from jax.experimental.pallas import tpu as pltpu
from jax.experimental.pallas import tpu_sc as plsc

_BINS = (1, 2, 3, 4, 8, 16, 32, 64, 128, 256, 384)

_B = 16384
_D = 20
_NC, _NS, _L = 2, 16, 16
_NW = _NC * _NS
_BPW = _B // _NW


def _body(len_hbm, tab_hbm, out_hbm, len_v, tab_v, out_v):
    wid = lax.axis_index("s") * _NC + lax.axis_index("c")
    base = wid * _BPW
    pltpu.sync_copy(len_hbm.at[pl.ds(base, _BPW)], len_v)
    pltpu.sync_copy(tab_hbm, tab_v)
    lane = lax.iota(jnp.int32, _L)
    zero = lane * 0
    dvec = [zero + d for d in range(_D)]

    def blk(j, carry):
        v = len_v[pl.ds(j * _L, _L)]
        idx = lax.shift_right_logical(_BINS[0] - v, 31)
        for b in _BINS[1:]:
            idx = idx + lax.shift_right_logical(b - v, 31)
        for d in range(_D):
            out_v[d, pl.ds(j * _L, _L)] = plsc.load_gather(tab_v, [idx, dvec[d]])
        return carry

    lax.fori_loop(0, _BPW // _L, blk, 0)
    pltpu.sync_copy(out_v, out_hbm.at[:, pl.ds(base, _BPW)])


def kernel(lengths, table):
    mesh = plsc.VectorSubcoreMesh(core_axis_name="c", subcore_axis_name="s")
    out_t = pl.kernel(
        _body,
        out_type=jax.ShapeDtypeStruct((_D, _B), jnp.float32),
        mesh=mesh,
        scratch_types=[
            pltpu.VMEM((_BPW,), jnp.int32),
            pltpu.VMEM((12, _D), jnp.float32),
            pltpu.VMEM((_D, _BPW), jnp.float32),
        ],
        compiler_params=pltpu.CompilerParams(needs_layout_passes=False),
    )(lengths, table)
    return jnp.swapaxes(out_t, 0, 1)


# speedup vs baseline: 1.4938x; 1.1967x over previous
import jax
import jax.numpy as jnp
from jax import lax
from jax.experimental import pallas as pl
from jax.experimental.pallas import tpu as pltpu
from jax.experimental.pallas import tpu_sc as plsc

_BINS = (1, 2, 3, 4, 8, 16, 32, 64, 128, 256, 384)

_B = 16384
_D = 20
_NC, _NS, _L = 2, 16, 16
_NW = _NC * _NS
_BPW = _B // _NW


_STR = 17


def _body(len_hbm, tab_hbm, out_hbm, len_v, tab_v, tabs_v, out_v):
    wid = lax.axis_index("s") * _NC + lax.axis_index("c")
    base = wid * _BPW
    pltpu.sync_copy(len_hbm.at[pl.ds(base, _BPW)], len_v)
    pltpu.sync_copy(tab_hbm, tab_v)
    for r in range(12):
        tabs_v[pl.ds(_STR * r, _L)] = tab_v[r, pl.ds(0, _L)]
        tabs_v[pl.ds(_STR * r + (_D - _L), _L)] = tab_v[r, pl.ds(_D - _L, _L)]

    def blk(j, carry):
        v = len_v[pl.ds(j * _L, _L)]
        idx = lax.shift_right_logical(_BINS[0] - v, 31)
        for b in _BINS[1:]:
            idx = idx + lax.shift_right_logical(b - v, 31)
        g = idx * _STR
        for d in range(_D):
            out_v[d, pl.ds(j * _L, _L)] = plsc.load_gather(tabs_v, [g + d])
        return carry

    lax.fori_loop(0, _BPW // _L, blk, 0)
    pltpu.sync_copy(out_v, out_hbm.at[:, pl.ds(base, _BPW)])


def kernel(lengths, table):
    mesh = plsc.VectorSubcoreMesh(core_axis_name="c", subcore_axis_name="s")
    out_t = pl.kernel(
        _body,
        out_type=jax.ShapeDtypeStruct((_D, _B), jnp.float32),
        mesh=mesh,
        scratch_types=[
            pltpu.VMEM((_BPW,), jnp.int32),
            pltpu.VMEM((12, _D), jnp.float32),
            pltpu.VMEM((224,), jnp.float32),
            pltpu.VMEM((_D, _BPW), jnp.float32),
        ],
        compiler_params=pltpu.CompilerParams(needs_layout_passes=False),
    )(lengths, table)
    return jnp.swapaxes(out_t, 0, 1)


# speedup vs baseline: 1.4947x; 1.0005x over previous
import jax
import jax.numpy as jnp
from jax import lax
from jax.experimental import pallas as pl
from jax.experimental.pallas import tpu as pltpu
from jax.experimental.pallas import tpu_sc as plsc

_BINS = (1, 2, 3, 4, 8, 16, 32, 64, 128, 256, 384)

_B = 16384
_D = 20
_NC, _NS, _L = 2, 16, 16
_NW = _NC * _NS
_BPW = _B // _NW


_STR = 21


def _body(len_hbm, tab_hbm, out_hbm, len_v, tab_v, tabs_v, out_v):
    wid = lax.axis_index("s") * _NC + lax.axis_index("c")
    base = wid * _BPW
    pltpu.sync_copy(len_hbm.at[pl.ds(base, _BPW)], len_v)
    pltpu.sync_copy(tab_hbm, tab_v)
    for r in range(12):
        tabs_v[pl.ds(_STR * r, _L)] = tab_v[r, pl.ds(0, _L)]
        tabs_v[pl.ds(_STR * r + (_D - _L), _L)] = tab_v[r, pl.ds(_D - _L, _L)]

    def blk(j, carry):
        v = len_v[pl.ds(j * _L, _L)]
        idx = lax.shift_right_logical(_BINS[0] - v, 31)
        for b in _BINS[1:]:
            idx = idx + lax.shift_right_logical(b - v, 31)
        g = idx * _STR
        for d in range(_D):
            out_v[d, pl.ds(j * _L, _L)] = plsc.load_gather(tabs_v, [g + d])
        return carry

    lax.fori_loop(0, _BPW // _L, blk, 0)
    pltpu.sync_copy(out_v, out_hbm.at[:, pl.ds(base, _BPW)])


def kernel(lengths, table):
    mesh = plsc.VectorSubcoreMesh(core_axis_name="c", subcore_axis_name="s")
    out_t = pl.kernel(
        _body,
        out_type=jax.ShapeDtypeStruct((_D, _B), jnp.float32),
        mesh=mesh,
        scratch_types=[
            pltpu.VMEM((_BPW,), jnp.int32),
            pltpu.VMEM((12, _D), jnp.float32),
            pltpu.VMEM((256,), jnp.float32),
            pltpu.VMEM((_D, _BPW), jnp.float32),
        ],
        compiler_params=pltpu.CompilerParams(needs_layout_passes=False),
    )(lengths, table)
    return jnp.swapaxes(out_t, 0, 1)
